# baseline (speedup 1.0000x reference)
import functools

import jax
import jax.numpy as jnp
from jax import lax
from jax.experimental import pallas as pl
from jax.experimental.pallas import tpu as pltpu
from jax.experimental.pallas import tpu_sc as plsc

NC = 2
NS = 16
NW = NC * NS
LANES = 16
IDXB = 128
CH_I = 8
CH_P = CH_I * IDXB
RW = 200


def _sc_partials(ind2w, inter, *, n_atoms, n_pairs):
    n_chunks = n_pairs // CH_P
    n_rchunks = n_atoms // RW
    PB = 2 * CH_P // IDXB

    mesh = plsc.VectorSubcoreMesh(core_axis_name="c", subcore_axis_name="s")

    @functools.partial(
        pl.kernel,
        out_type=jax.ShapeDtypeStruct((NC, n_atoms, LANES), jnp.float32),
        mesh=mesh,
        scratch_types=[
            pltpu.VMEM((PB * IDXB,), jnp.int32),
            pltpu.VMEM((CH_I, IDXB), jnp.int32),
            pltpu.VMEM((CH_P, LANES), jnp.float32),
            pltpu.VMEM_SHARED((n_atoms, LANES), jnp.float32),
            pltpu.SemaphoreType.DMA,
        ],
        compiler_params=pltpu.CompilerParams(use_tc_tiling_on_sc=False),
    )
    def body(ind2_hbm, inter_hbm, out_hbm, pbuf, idxv, rows, acc, sem):
        c = lax.axis_index("c")
        s = lax.axis_index("s")
        w = s * NC + c

        def zero_row(i, _):
            rows[i] = jnp.zeros((LANES,), jnp.float32)
            return 0
        lax.fori_loop(0, RW, zero_row, 0)

        n_z = (n_rchunks - s + NS - 1) // NS

        def zero_chunk(z, _):
            zc = s + z * NS
            pltpu.sync_copy(rows.at[pl.ds(0, RW)], acc.at[pl.ds(zc * RW, RW)])
            return 0

        lax.fori_loop(0, n_z, zero_chunk, 0)
        plsc.subcore_barrier()

        n_k = (n_chunks - w + NW - 1) // NW

        lane = jnp.arange(LANES, dtype=jnp.int32)

        def chunk_body(k, _):
            cid = w + k * NW
            ld_p = pltpu.async_copy(
                ind2_hbm.at[pl.ds(cid * 2 * CH_P, 2 * CH_P)], pbuf, sem)
            ld_r = pltpu.async_copy(inter_hbm.at[pl.ds(cid * CH_P, CH_P)],
                                    rows, sem)
            ld_p.wait()
            ev = (2 * lane) % LANES
            lo = lane < (LANES // 2)
            for j in range(CH_I):
                for v in range(IDXB // LANES):
                    base = (j * IDXB + v * LANES) * 2
                    v0 = pbuf[pl.ds(base, LANES)]
                    v1 = pbuf[pl.ds(base + LANES, LANES)]
                    g0 = v0[ev]
                    g1 = v1[ev]
                    idxv[j, pl.ds(v * LANES, LANES)] = jnp.where(lo, g0, g1)
            ld_r.wait()
            descs = [
                pltpu.async_copy(rows.at[pl.ds(j * IDXB, IDXB)],
                                 acc.at[idxv.at[j]], sem, add=True)
                for j in range(CH_I)
            ]
            for dsc in descs:
                dsc.wait()
            return 0

        lax.fori_loop(0, n_k, chunk_body, 0)
        plsc.subcore_barrier()

        def dump_chunk(z, _):
            zc = s + z * NS
            pltpu.sync_copy(acc.at[pl.ds(zc * RW, RW)],
                            out_hbm.at[c, pl.ds(zc * RW, RW)])
            return 0

        lax.fori_loop(0, n_z, dump_chunk, 0)

    return body(ind2w, inter)


def _merge_body(p_ref, o_ref):
    o_ref[...] = p_ref[0] + p_ref[1]


def kernel(ind_2, prop, inter):
    n_atoms = prop.shape[0]
    n_pairs, n_inter = inter.shape
    assert n_inter == LANES
    assert n_pairs % CH_P == 0
    assert n_atoms % RW == 0
    assert (n_atoms * LANES) % 128 == 0

    ind2w = ind_2.reshape(n_pairs * 2)
    partials = _sc_partials(ind2w, inter, n_atoms=n_atoms, n_pairs=n_pairs)

    wide = n_atoms * LANES // 128
    pr = partials.reshape(NC, wide, 128)
    merged = pl.pallas_call(
        _merge_body,
        out_shape=jax.ShapeDtypeStruct((wide, 128), jnp.float32),
    )(pr)
    return merged.reshape(n_atoms, LANES)


# speedup vs baseline: 3.8371x; 3.8371x over previous
import functools

import jax
import jax.numpy as jnp
from jax import lax
from jax.experimental import pallas as pl
from jax.experimental.pallas import tpu as pltpu
from jax.experimental.pallas import tpu_sc as plsc

NC = 2
NS = 16
NW = NC * NS
LANES = 16
IDXB = 128
CH_I = 8
CH_P = CH_I * IDXB
RW = 200


def _sc_partials(idx2d, inter, *, n_atoms, n_pairs):
    n_chunks = n_pairs // CH_P
    n_rchunks = n_atoms // RW

    mesh = plsc.VectorSubcoreMesh(core_axis_name="c", subcore_axis_name="s")

    @functools.partial(
        pl.kernel,
        out_type=jax.ShapeDtypeStruct((NC, n_atoms, LANES), jnp.float32),
        mesh=mesh,
        scratch_types=[
            pltpu.VMEM((CH_I, 1, IDXB), jnp.int32),
            pltpu.VMEM((CH_P, LANES), jnp.float32),
            pltpu.VMEM((RW, LANES), jnp.float32),
            pltpu.VMEM_SHARED((n_atoms, LANES), jnp.float32),
            pltpu.SemaphoreType.DMA,
        ],
        compiler_params=pltpu.CompilerParams(use_tc_tiling_on_sc=False),
    )
    def body(idx_hbm, inter_hbm, out_hbm, idxv, rows, zbuf, acc, sem):
        c = lax.axis_index("c")
        s = lax.axis_index("s")
        w = s * NC + c

        def zero_row(i, _):
            zbuf[i] = jnp.zeros((LANES,), jnp.float32)
            return 0
        lax.fori_loop(0, RW, zero_row, 0)

        n_z = (n_rchunks - s + NS - 1) // NS

        def zero_chunk(z, _):
            zc = s + z * NS
            pltpu.sync_copy(zbuf, acc.at[pl.ds(zc * RW, RW)])
            return 0

        lax.fori_loop(0, n_z, zero_chunk, 0)
        plsc.subcore_barrier()

        n_k = (n_chunks - w + NW - 1) // NW

        lane = jnp.arange(LANES, dtype=jnp.int32)

        def chunk_body(k, _):
            cid = w + k * NW
            ld_p = pltpu.async_copy(
                idx_hbm.at[pl.ds(cid * CH_I, CH_I), pl.ds(0, 1)], idxv, sem)
            ld_r = pltpu.async_copy(
                inter_hbm.at[pl.ds(cid * CH_P, CH_P)], rows, sem)
            ld_p.wait()
            ld_r.wait()
            descs = [
                pltpu.async_copy(rows.at[pl.ds(j * IDXB, IDXB)],
                                 acc.at[idxv.at[j, 0]], sem, add=True)
                for j in range(CH_I)
            ]
            for dsc in descs:
                dsc.wait()
            return 0

        lax.fori_loop(0, n_k, chunk_body, 0)
        plsc.subcore_barrier()

        def dump_chunk(z, _):
            zc = s + z * NS
            pltpu.sync_copy(acc.at[pl.ds(zc * RW, RW)],
                            out_hbm.at[c, pl.ds(zc * RW, RW)])
            return 0

        lax.fori_loop(0, n_z, dump_chunk, 0)

    return body(idx2d, inter)


def _merge_body(p_ref, o_ref):
    o_ref[...] = p_ref[0] + p_ref[1]


def kernel(ind_2, prop, inter):
    n_atoms = prop.shape[0]
    n_pairs, n_inter = inter.shape
    assert n_inter == LANES
    assert n_pairs % CH_P == 0
    assert n_atoms % RW == 0
    assert (n_atoms * LANES) % 128 == 0

    idx3d = ind_2.reshape(n_pairs // IDXB, IDXB, 2).transpose(0, 2, 1)
    partials = _sc_partials(idx3d, inter, n_atoms=n_atoms, n_pairs=n_pairs)

    wide = n_atoms * LANES // 128
    pr = partials.reshape(NC, wide, 128)
    merged = pl.pallas_call(
        _merge_body,
        out_shape=jax.ShapeDtypeStruct((wide, 128), jnp.float32),
    )(pr)
    return merged.reshape(n_atoms, LANES)


# speedup vs baseline: 3.8382x; 1.0003x over previous
import functools

import jax
import jax.numpy as jnp
from jax import lax
from jax.experimental import pallas as pl
from jax.experimental.pallas import tpu as pltpu
from jax.experimental.pallas import tpu_sc as plsc

NC = 2
NS = 16
NW = NC * NS
LANES = 16
IDXB = 128
CH_I = 8
CH_P = CH_I * IDXB
RW = 200


def _sc_partials(idx2d, inter, *, n_atoms, n_pairs):
    n_chunks = n_pairs // CH_P
    n_rchunks = n_atoms // RW

    mesh = plsc.VectorSubcoreMesh(core_axis_name="c", subcore_axis_name="s")

    @functools.partial(
        pl.kernel,
        out_type=jax.ShapeDtypeStruct((NC, n_atoms, LANES), jnp.float32),
        mesh=mesh,
        scratch_types=[
            pltpu.VMEM((CH_I, 1, IDXB), jnp.int32),
            pltpu.VMEM((CH_I, IDXB, LANES), jnp.float32),
            pltpu.VMEM((RW, LANES), jnp.float32),
            pltpu.VMEM_SHARED((n_atoms, LANES), jnp.float32),
            pltpu.SemaphoreType.DMA,
        ],
        compiler_params=pltpu.CompilerParams(use_tc_tiling_on_sc=False),
    )
    def body(idx_hbm, inter_hbm, out_hbm, idxv, rows, zbuf, acc, sem):
        c = lax.axis_index("c")
        s = lax.axis_index("s")
        w = s * NC + c

        def zero_row(i, _):
            zbuf[i] = jnp.zeros((LANES,), jnp.float32)
            return 0
        lax.fori_loop(0, RW, zero_row, 0)

        n_z = (n_rchunks - s + NS - 1) // NS

        def zero_chunk(z, _):
            zc = s + z * NS
            pltpu.sync_copy(zbuf, acc.at[pl.ds(zc * RW, RW)])
            return 0

        lax.fori_loop(0, n_z, zero_chunk, 0)
        plsc.subcore_barrier()

        n_k = (n_chunks - w + NW - 1) // NW

        lane = jnp.arange(LANES, dtype=jnp.int32)

        def chunk_body(k, _):
            cid = w + k * NW
            ld_p = pltpu.async_copy(
                idx_hbm.at[pl.ds(cid * CH_I, CH_I), pl.ds(0, 1)], idxv, sem)
            ld_r = pltpu.async_copy(
                inter_hbm.at[pl.ds(cid * CH_I, CH_I)], rows, sem)
            ld_p.wait()
            ld_r.wait()
            descs = [
                pltpu.async_copy(rows.at[j],
                                 acc.at[idxv.at[j, 0]], sem, add=True)
                for j in range(CH_I)
            ]
            for dsc in descs:
                dsc.wait()
            return 0

        lax.fori_loop(0, n_k, chunk_body, 0)
        plsc.subcore_barrier()

        def dump_chunk(z, _):
            zc = s + z * NS
            pltpu.sync_copy(acc.at[pl.ds(zc * RW, RW)],
                            out_hbm.at[c, pl.ds(zc * RW, RW)])
            return 0

        lax.fori_loop(0, n_z, dump_chunk, 0)

    return body(idx2d, inter)


def _merge_body(p_ref, o_ref):
    o_ref[...] = p_ref[0] + p_ref[1]


def kernel(ind_2, prop, inter):
    n_atoms = prop.shape[0]
    n_pairs, n_inter = inter.shape
    assert n_inter == LANES
    assert n_pairs % CH_P == 0
    assert n_atoms % RW == 0
    assert (n_atoms * LANES) % 128 == 0

    idx3d = ind_2.reshape(n_pairs // IDXB, IDXB, 2).transpose(0, 2, 1)
    inter_p = inter.reshape(n_pairs // IDXB, IDXB, LANES)
    partials = _sc_partials(idx3d, inter_p, n_atoms=n_atoms, n_pairs=n_pairs)

    wide = n_atoms * LANES // 128
    pr = partials.reshape(NC, wide, 128)
    merged = pl.pallas_call(
        _merge_body,
        out_shape=jax.ShapeDtypeStruct((wide, 128), jnp.float32),
    )(pr)
    return merged.reshape(n_atoms, LANES)
